# baseline (speedup 1.0000x reference)
import dataclasses
import functools

import jax
import jax.numpy as jnp
from jax import lax
from jax.experimental import pallas as pl
from jax.experimental.pallas import tpu as pltpu
from jax.experimental.pallas import tpu_sc as plsc

W, H, D = 32, 32, 32
G = W * H * D
N, C = 100000, 128

NC, NS = 2, 16
NW = NC * NS
NP = 102400
SL = NP // NW
NV = SL // 16
GB = 512
NRANGE = G // GB
WSORT = SL + NRANGE * 7
SEG = 256
WCAP = WSORT + SEG
B = 256

_mesh = plsc.VectorSubcoreMesh(core_axis_name="c", subcore_axis_name="s")


def _compiler_params():
    cp = pltpu.CompilerParams()
    if "needs_layout_passes" in pltpu.CompilerParams.__dataclass_fields__:
        cp = dataclasses.replace(cp, needs_layout_passes=False)
    return cp


_GDN = lax.GatherDimensionNumbers(
    offset_dims=(), collapsed_slice_dims=(0,), start_index_map=(0,))


def _bcast_lane(vec, lane):
    idx = jnp.broadcast_to(lane.astype(jnp.int32), (16,))[:, None]
    return lax.gather(vec, idx, _GDN, (1,),
                      mode=lax.GatherScatterMode.PROMISE_IN_BOUNDS)


@functools.partial(
    pl.kernel,
    out_type=[
        jax.ShapeDtypeStruct((NW * WCAP,), jnp.int32),
        jax.ShapeDtypeStruct((NW * NRANGE,), jnp.int32),
        jax.ShapeDtypeStruct((NW * NRANGE,), jnp.int32),
    ],
    mesh=_mesh,
    scratch_types=[
        pltpu.VMEM((3, SL), jnp.float32),
        pltpu.VMEM((SL,), jnp.int32),
        pltpu.VMEM((WSORT,), jnp.int32),
        pltpu.VMEM((NRANGE,), jnp.int32),
        pltpu.VMEM((NRANGE,), jnp.int32),
        pltpu.VMEM((NRANGE,), jnp.int32),
    ],
    compiler_params=_compiler_params(),
)
def _route(pts_hbm, buckets_hbm, counts_hbm, offs_hbm,
           pbuf, cellb, stag, hist, offs, cur):
    wid = lax.axis_index("s") * NC + lax.axis_index("c")
    base = wid * SL
    pltpu.sync_copy(pts_hbm.at[:, pl.ds(base, SL)], pbuf)
    iota = lax.iota(jnp.int32, 16)
    ones = jnp.ones((16,), jnp.int32)

    @pl.loop(0, NRANGE // 16)
    def _(i):
        hist[pl.ds(i * 16, 16)] = jnp.zeros((16,), jnp.int32)

    @pl.loop(0, NV)
    def _(v):
        off = v * 16
        x = pbuf[0, pl.ds(off, 16)]
        y = pbuf[1, pl.ds(off, 16)]
        z = pbuf[2, pl.ds(off, 16)]
        ix = jnp.clip((x * W).astype(jnp.int32), 0, W - 1)
        iy = jnp.clip((y * H).astype(jnp.int32), 0, H - 1)
        iz = jnp.clip((z * D).astype(jnp.int32), 0, D - 1)
        flat = (ix * H + iy) * D + iz
        m = (base + off + iota) < N
        cellb[pl.ds(off, 16)] = flat
        plsc.addupdate_scatter(hist, [flat >> 9], ones, mask=m)

    carry = jnp.zeros((16,), jnp.int32)
    for t in range(NRANGE // 16):
        h = hist[pl.ds(t * 16, 16)]
        hp = (h + 7) & jnp.int32(-8)
        inc = plsc.cumsum(hp)
        off_v = inc - hp + carry
        offs[pl.ds(t * 16, 16)] = off_v
        cur[pl.ds(t * 16, 16)] = off_v
        carry = carry + _bcast_lane(inc, jnp.int32(15))

    @pl.loop(0, NV)
    def _(v):
        off = v * 16
        flat = cellb[pl.ds(off, 16)]
        pid = base + off + iota
        m = pid < N
        bucket = flat >> 9
        val = (pid << 9) | (flat & 511)
        dup, _ = plsc.scan_count(bucket, mask=m)
        c0 = plsc.load_gather(cur, [bucket])
        pos = c0 + dup - 1
        plsc.store_scatter(stag, [pos], val, mask=m)
        plsc.addupdate_scatter(cur, [bucket], ones, mask=m)

    pltpu.sync_copy(stag, buckets_hbm.at[pl.ds(wid * WCAP, WSORT)])
    pltpu.sync_copy(hist, counts_hbm.at[pl.ds(wid * NRANGE, NRANGE)])
    pltpu.sync_copy(offs, offs_hbm.at[pl.ds(wid * NRANGE, NRANGE)])


@functools.partial(
    pl.kernel,
    out_type=jax.ShapeDtypeStruct((G * C,), jnp.float32),
    mesh=_mesh,
    scratch_types=[
        pltpu.VMEM((GB * C,), jnp.float32),
        pltpu.VMEM((B, C), jnp.float32),
        pltpu.VMEM((B,), jnp.int32),
        pltpu.VMEM((B + 16,), jnp.int32),
        pltpu.VMEM((NW * SEG,), jnp.int32),
        pltpu.VMEM((SEG,), jnp.int32),
        pltpu.VMEM((NW * NRANGE,), jnp.int32),
        pltpu.VMEM((NW * NRANGE,), jnp.int32),
        pltpu.SemaphoreType.DMA,
        pltpu.SemaphoreType.DMA,
    ],
    compiler_params=_compiler_params(),
)
def _pool(buckets_hbm, counts_hbm, offs_hbm, feat_hbm, out_hbm,
          acc, rows, mpid, mcell, segs, xbuf, cntb, offb, fsem, ssem):
    wid = lax.axis_index("s") * NC + lax.axis_index("c")
    iota = lax.iota(jnp.int32, 16)
    zeros16 = jnp.zeros((16,), jnp.float32)

    pltpu.sync_copy(counts_hbm, cntb)
    pltpu.sync_copy(offs_hbm, offb)

    @pl.loop(0, B // 16)
    def _(i):
        mpid[pl.ds(i * 16, 16)] = jnp.zeros((16,), jnp.int32)
        mcell[pl.ds(i * 16, 16)] = jnp.zeros((16,), jnp.int32)

    def read_at(buf, w, rng):
        idx = w * NRANGE + rng
        grp = (idx >> 4) << 4
        vec = buf[pl.ds(grp, 16)]
        sel = jnp.where(iota == (idx & 15), vec, jnp.int32(0))
        return jnp.max(sel)

    def flush(cnt):
        gathers = [
            pltpu.async_copy(feat_hbm.at[mpid.at[pl.ds(h * 128, 128)]],
                             rows.at[pl.ds(h * 128, 128)], fsem)
            for h in range(B // 128)
        ]
        for g_ in gathers:
            g_.wait()

        def row_body(r, carry):
            rbase = mcell[pl.ds(r, 16)][0] * C
            for j in range(C // 16):
                sl = pl.ds(rbase + j * 16, 16)
                acc[sl] = jnp.maximum(acc[sl], rows[r, pl.ds(j * 16, 16)])
            return carry

        lax.fori_loop(0, cnt, row_body, 0)
        return jnp.int32(0)

    for rnd in range(NRANGE // NW):
        rng = wid * (NRANGE // NW) + rnd

        @plsc.parallel_loop(0, GB * C // 16, unroll=8)
        def _(i):
            acc[pl.ds(i * 16, 16)] = zeros16

        heads = [
            pltpu.async_copy(
                buckets_hbm.at[pl.ds(
                    pl.multiple_of(w * WCAP + read_at(offb, w, rng), 8),
                    SEG)],
                segs.at[pl.ds(w * SEG, SEG)], ssem)
            for w in range(NW)
        ]
        for h_ in heads:
            h_.wait()

        def append_chunk(load_fn, rem, cnt):
            nvec = (rem + 15) >> 4

            def vec_body(v, cnt):
                cnt = lax.cond(cnt > B - 16, flush, lambda c: c, cnt)
                vals = load_fn(v)
                mm = (v * 16 + iota) < rem
                plsc.store_compressed(mpid.at[pl.ds(cnt, 16)], vals >> 9,
                                      mask=mm)
                plsc.store_compressed(mcell.at[pl.ds(cnt, 16)], vals & 511,
                                      mask=mm)
                return cnt + jnp.minimum(jnp.int32(16), rem - v * 16)

            return lax.fori_loop(0, nvec, vec_body, cnt)

        def seg_body(w, cnt):
            cw = read_at(cntb, w, rng)
            cnt = append_chunk(lambda v: segs[pl.ds(w * SEG + v * 16, 16)],
                               jnp.minimum(cw, jnp.int32(SEG)), cnt)

            def over_body(k, cnt):
                pltpu.sync_copy(
                    buckets_hbm.at[pl.ds(
                        pl.multiple_of(
                            w * WCAP + read_at(offb, w, rng) + k * SEG, 8),
                        SEG)],
                    xbuf)
                rem = jnp.minimum(cw - k * SEG, jnp.int32(SEG))
                return append_chunk(lambda v: xbuf[pl.ds(v * 16, 16)],
                                    rem, cnt)

            nseg = (cw + SEG - 1) >> 8
            return lax.fori_loop(1, nseg, over_body, cnt)

        cnt = lax.fori_loop(0, NW, seg_body, jnp.int32(0))
        flush(cnt)
        pltpu.sync_copy(acc, out_hbm.at[pl.ds(rng * GB * C, GB * C)])


def kernel(features, points):
    pts_t = jnp.zeros((3, NP), jnp.float32).at[:, :N].set(points.T)
    buckets, counts, offs = _route(pts_t)
    out_flat = _pool(buckets, counts, offs, features)
    return out_flat.reshape(G, C)


# speedup vs baseline: 1.0587x; 1.0587x over previous
import dataclasses
import functools

import jax
import jax.numpy as jnp
from jax import lax
from jax.experimental import pallas as pl
from jax.experimental.pallas import tpu as pltpu
from jax.experimental.pallas import tpu_sc as plsc

W, H, D = 32, 32, 32
G = W * H * D
N, C = 100000, 128

NC, NS = 2, 16
NW = NC * NS
NP = 102400
SL = NP // NW
NV = SL // 16
GB = 512
NRANGE = G // GB
WSORT = SL + NRANGE * 7
SEG = 256
WCAP = WSORT + SEG
B = 256

_mesh = plsc.VectorSubcoreMesh(core_axis_name="c", subcore_axis_name="s")


def _compiler_params():
    cp = pltpu.CompilerParams()
    if "needs_layout_passes" in pltpu.CompilerParams.__dataclass_fields__:
        cp = dataclasses.replace(cp, needs_layout_passes=False)
    return cp


_GDN = lax.GatherDimensionNumbers(
    offset_dims=(), collapsed_slice_dims=(0,), start_index_map=(0,))


def _bcast_lane(vec, lane):
    idx = jnp.broadcast_to(lane.astype(jnp.int32), (16,))[:, None]
    return lax.gather(vec, idx, _GDN, (1,),
                      mode=lax.GatherScatterMode.PROMISE_IN_BOUNDS)


@functools.partial(
    pl.kernel,
    out_type=[
        jax.ShapeDtypeStruct((NW * WCAP,), jnp.int32),
        jax.ShapeDtypeStruct((NW * NRANGE,), jnp.int32),
        jax.ShapeDtypeStruct((NW * NRANGE,), jnp.int32),
    ],
    mesh=_mesh,
    scratch_types=[
        pltpu.VMEM((3, SL), jnp.float32),
        pltpu.VMEM((SL,), jnp.int32),
        pltpu.VMEM((WSORT,), jnp.int32),
        pltpu.VMEM((NRANGE,), jnp.int32),
        pltpu.VMEM((NRANGE,), jnp.int32),
        pltpu.VMEM((NRANGE,), jnp.int32),
    ],
    compiler_params=_compiler_params(),
)
def _route(pts_hbm, buckets_hbm, counts_hbm, offs_hbm,
           pbuf, cellb, stag, hist, offs, cur):
    wid = lax.axis_index("s") * NC + lax.axis_index("c")
    base = wid * SL
    pltpu.sync_copy(pts_hbm.at[:, pl.ds(base, SL)], pbuf)
    iota = lax.iota(jnp.int32, 16)
    ones = jnp.ones((16,), jnp.int32)

    @pl.loop(0, NRANGE // 16)
    def _(i):
        hist[pl.ds(i * 16, 16)] = jnp.zeros((16,), jnp.int32)

    @pl.loop(0, NV)
    def _(v):
        off = v * 16
        x = pbuf[0, pl.ds(off, 16)]
        y = pbuf[1, pl.ds(off, 16)]
        z = pbuf[2, pl.ds(off, 16)]
        ix = jnp.clip((x * W).astype(jnp.int32), 0, W - 1)
        iy = jnp.clip((y * H).astype(jnp.int32), 0, H - 1)
        iz = jnp.clip((z * D).astype(jnp.int32), 0, D - 1)
        flat = (ix * H + iy) * D + iz
        m = (base + off + iota) < N
        cellb[pl.ds(off, 16)] = flat
        plsc.addupdate_scatter(hist, [flat >> 9], ones, mask=m)

    carry = jnp.zeros((16,), jnp.int32)
    for t in range(NRANGE // 16):
        h = hist[pl.ds(t * 16, 16)]
        hp = (h + 7) & jnp.int32(-8)
        inc = plsc.cumsum(hp)
        off_v = inc - hp + carry
        offs[pl.ds(t * 16, 16)] = off_v
        cur[pl.ds(t * 16, 16)] = off_v
        carry = carry + _bcast_lane(inc, jnp.int32(15))

    @pl.loop(0, NV)
    def _(v):
        off = v * 16
        flat = cellb[pl.ds(off, 16)]
        pid = base + off + iota
        m = pid < N
        bucket = flat >> 9
        val = (pid << 9) | (flat & 511)
        dup, _ = plsc.scan_count(bucket, mask=m)
        c0 = plsc.load_gather(cur, [bucket])
        pos = c0 + dup - 1
        plsc.store_scatter(stag, [pos], val, mask=m)
        plsc.addupdate_scatter(cur, [bucket], ones, mask=m)

    pltpu.sync_copy(stag, buckets_hbm.at[pl.ds(wid * WCAP, WSORT)])
    pltpu.sync_copy(hist, counts_hbm.at[pl.ds(wid * NRANGE, NRANGE)])
    pltpu.sync_copy(offs, offs_hbm.at[pl.ds(wid * NRANGE, NRANGE)])


@functools.partial(
    pl.kernel,
    out_type=jax.ShapeDtypeStruct((G * C,), jnp.float32),
    mesh=_mesh,
    scratch_types=[
        pltpu.VMEM((GB * C,), jnp.float32),
        pltpu.VMEM((B, C), jnp.float32),
        pltpu.VMEM((B,), jnp.int32),
        pltpu.VMEM((B + 16,), jnp.int32),
        pltpu.VMEM((NW * SEG,), jnp.int32),
        pltpu.VMEM((SEG,), jnp.int32),
        pltpu.VMEM((NW * NRANGE,), jnp.int32),
        pltpu.VMEM((NW * NRANGE,), jnp.int32),
        pltpu.SemaphoreType.DMA,
        pltpu.SemaphoreType.DMA,
    ],
    compiler_params=_compiler_params(),
)
def _pool(buckets_hbm, counts_hbm, offs_hbm, feat_hbm, out_hbm,
          acc, rows, mpid, mcell, segs, xbuf, cntb, offb, fsem, ssem):
    wid = lax.axis_index("s") * NC + lax.axis_index("c")
    iota = lax.iota(jnp.int32, 16)
    zeros16 = jnp.zeros((16,), jnp.float32)

    pltpu.sync_copy(counts_hbm, cntb)
    pltpu.sync_copy(offs_hbm, offb)

    @pl.loop(0, B // 16)
    def _(i):
        mpid[pl.ds(i * 16, 16)] = jnp.zeros((16,), jnp.int32)
        mcell[pl.ds(i * 16, 16)] = jnp.zeros((16,), jnp.int32)

    def read_at(buf, w, rng):
        idx = w * NRANGE + rng
        grp = (idx >> 4) << 4
        vec = buf[pl.ds(grp, 16)]
        sel = jnp.where(iota == (idx & 15), vec, jnp.int32(0))
        return jnp.max(sel)

    def flush(cnt):
        gathers = [
            pltpu.async_copy(feat_hbm.at[mpid.at[pl.ds(h * 128, 128)]],
                             rows.at[pl.ds(h * 128, 128)], fsem)
            for h in range(B // 128)
        ]

        def row_body(r, carry):
            rbase = mcell[pl.ds(r, 16)][0] * C
            for j in range(C // 16):
                sl = pl.ds(rbase + j * 16, 16)
                acc[sl] = jnp.maximum(acc[sl], rows[r, pl.ds(j * 16, 16)])
            return carry

        gathers[0].wait()
        lax.fori_loop(0, jnp.minimum(cnt, jnp.int32(128)), row_body, 0)
        gathers[1].wait()
        lax.fori_loop(jnp.int32(128), cnt, row_body, 0)
        return jnp.int32(0)

    for rnd in range(NRANGE // NW):
        rng = wid * (NRANGE // NW) + rnd

        @plsc.parallel_loop(0, GB * C // 16, unroll=8)
        def _(i):
            acc[pl.ds(i * 16, 16)] = zeros16

        heads = [
            pltpu.async_copy(
                buckets_hbm.at[pl.ds(
                    pl.multiple_of(w * WCAP + read_at(offb, w, rng), 8),
                    SEG)],
                segs.at[pl.ds(w * SEG, SEG)], ssem)
            for w in range(NW)
        ]
        for h_ in heads:
            h_.wait()

        def append_chunk(load_fn, rem, cnt):
            nvec = (rem + 15) >> 4

            def vec_body(v, cnt):
                cnt = lax.cond(cnt > B - 16, flush, lambda c: c, cnt)
                vals = load_fn(v)
                mm = (v * 16 + iota) < rem
                plsc.store_compressed(mpid.at[pl.ds(cnt, 16)], vals >> 9,
                                      mask=mm)
                plsc.store_compressed(mcell.at[pl.ds(cnt, 16)], vals & 511,
                                      mask=mm)
                return cnt + jnp.minimum(jnp.int32(16), rem - v * 16)

            return lax.fori_loop(0, nvec, vec_body, cnt)

        def seg_body(w, cnt):
            cw = read_at(cntb, w, rng)
            cnt = append_chunk(lambda v: segs[pl.ds(w * SEG + v * 16, 16)],
                               jnp.minimum(cw, jnp.int32(SEG)), cnt)

            def over_body(k, cnt):
                pltpu.sync_copy(
                    buckets_hbm.at[pl.ds(
                        pl.multiple_of(
                            w * WCAP + read_at(offb, w, rng) + k * SEG, 8),
                        SEG)],
                    xbuf)
                rem = jnp.minimum(cw - k * SEG, jnp.int32(SEG))
                return append_chunk(lambda v: xbuf[pl.ds(v * 16, 16)],
                                    rem, cnt)

            nseg = (cw + SEG - 1) >> 8
            return lax.fori_loop(1, nseg, over_body, cnt)

        cnt = lax.fori_loop(0, NW, seg_body, jnp.int32(0))
        flush(cnt)
        pltpu.sync_copy(acc, out_hbm.at[pl.ds(rng * GB * C, GB * C)])


def kernel(features, points):
    pts_t = jnp.zeros((3, NP), jnp.float32).at[:, :N].set(points.T)
    buckets, counts, offs = _route(pts_t)
    out_flat = _pool(buckets, counts, offs, features)
    return out_flat.reshape(G, C)


# speedup vs baseline: 1.0825x; 1.0224x over previous
import dataclasses
import functools

import jax
import jax.numpy as jnp
from jax import lax
from jax.experimental import pallas as pl
from jax.experimental.pallas import tpu as pltpu
from jax.experimental.pallas import tpu_sc as plsc

W, H, D = 32, 32, 32
G = W * H * D
N, C = 100000, 128

NC, NS = 2, 16
NW = NC * NS
NP = 102400
SL = NP // NW
NV = SL // 16
GB = 512
NRANGE = G // GB
WSORT = SL + NRANGE * 7
SEG = 256
WCAP = WSORT + SEG
B = 256

_mesh = plsc.VectorSubcoreMesh(core_axis_name="c", subcore_axis_name="s")


def _compiler_params():
    cp = pltpu.CompilerParams()
    if "needs_layout_passes" in pltpu.CompilerParams.__dataclass_fields__:
        cp = dataclasses.replace(cp, needs_layout_passes=False)
    return cp


_GDN = lax.GatherDimensionNumbers(
    offset_dims=(), collapsed_slice_dims=(0,), start_index_map=(0,))


def _bcast_lane(vec, lane):
    idx = jnp.broadcast_to(lane.astype(jnp.int32), (16,))[:, None]
    return lax.gather(vec, idx, _GDN, (1,),
                      mode=lax.GatherScatterMode.PROMISE_IN_BOUNDS)


@functools.partial(
    pl.kernel,
    out_type=[
        jax.ShapeDtypeStruct((NW * WCAP,), jnp.int32),
        jax.ShapeDtypeStruct((NW * NRANGE,), jnp.int32),
        jax.ShapeDtypeStruct((NW * NRANGE,), jnp.int32),
    ],
    mesh=_mesh,
    scratch_types=[
        pltpu.VMEM((3, SL), jnp.float32),
        pltpu.VMEM((SL,), jnp.int32),
        pltpu.VMEM((WSORT,), jnp.int32),
        pltpu.VMEM((NRANGE,), jnp.int32),
        pltpu.VMEM((NRANGE,), jnp.int32),
        pltpu.VMEM((NRANGE,), jnp.int32),
    ],
    compiler_params=_compiler_params(),
)
def _route(pts_hbm, buckets_hbm, counts_hbm, offs_hbm,
           pbuf, cellb, stag, hist, offs, cur):
    wid = lax.axis_index("s") * NC + lax.axis_index("c")
    base = wid * SL
    pltpu.sync_copy(pts_hbm.at[:, pl.ds(base, SL)], pbuf)
    iota = lax.iota(jnp.int32, 16)
    ones = jnp.ones((16,), jnp.int32)

    @pl.loop(0, NRANGE // 16)
    def _(i):
        hist[pl.ds(i * 16, 16)] = jnp.zeros((16,), jnp.int32)

    @pl.loop(0, NV)
    def _(v):
        off = v * 16
        x = pbuf[0, pl.ds(off, 16)]
        y = pbuf[1, pl.ds(off, 16)]
        z = pbuf[2, pl.ds(off, 16)]
        ix = jnp.clip((x * W).astype(jnp.int32), 0, W - 1)
        iy = jnp.clip((y * H).astype(jnp.int32), 0, H - 1)
        iz = jnp.clip((z * D).astype(jnp.int32), 0, D - 1)
        flat = (ix * H + iy) * D + iz
        m = (base + off + iota) < N
        cellb[pl.ds(off, 16)] = flat
        plsc.addupdate_scatter(hist, [flat >> 9], ones, mask=m)

    carry = jnp.zeros((16,), jnp.int32)
    for t in range(NRANGE // 16):
        h = hist[pl.ds(t * 16, 16)]
        hp = (h + 7) & jnp.int32(-8)
        inc = plsc.cumsum(hp)
        off_v = inc - hp + carry
        offs[pl.ds(t * 16, 16)] = off_v
        cur[pl.ds(t * 16, 16)] = off_v
        carry = carry + _bcast_lane(inc, jnp.int32(15))

    @pl.loop(0, NV)
    def _(v):
        off = v * 16
        flat = cellb[pl.ds(off, 16)]
        pid = base + off + iota
        m = pid < N
        bucket = flat >> 9
        val = (pid << 9) | (flat & 511)
        dup, _ = plsc.scan_count(bucket, mask=m)
        c0 = plsc.load_gather(cur, [bucket])
        pos = c0 + dup - 1
        plsc.store_scatter(stag, [pos], val, mask=m)
        plsc.addupdate_scatter(cur, [bucket], ones, mask=m)

    pltpu.sync_copy(stag, buckets_hbm.at[pl.ds(wid * WCAP, WSORT)])
    pltpu.sync_copy(hist, counts_hbm.at[pl.ds(wid * NRANGE, NRANGE)])
    pltpu.sync_copy(offs, offs_hbm.at[pl.ds(wid * NRANGE, NRANGE)])


@functools.partial(
    pl.kernel,
    out_type=jax.ShapeDtypeStruct((G * C,), jnp.float32),
    mesh=_mesh,
    scratch_types=[
        pltpu.VMEM((GB * C,), jnp.float32),
        pltpu.VMEM((B, C), jnp.float32),
        pltpu.VMEM((B,), jnp.int32),
        pltpu.VMEM((B + 16,), jnp.int32),
        pltpu.VMEM((NW * SEG,), jnp.int32),
        pltpu.VMEM((SEG,), jnp.int32),
        pltpu.VMEM((NW * NRANGE,), jnp.int32),
        pltpu.VMEM((NW * NRANGE,), jnp.int32),
        pltpu.SemaphoreType.DMA,
        pltpu.SemaphoreType.DMA,
    ],
    compiler_params=_compiler_params(),
)
def _pool(buckets_hbm, counts_hbm, offs_hbm, feat_hbm, out_hbm,
          acc, rows, mpid, mcell, segs, xbuf, cntb, offb, fsem, ssem):
    wid = lax.axis_index("s") * NC + lax.axis_index("c")
    iota = lax.iota(jnp.int32, 16)
    zeros16 = jnp.zeros((16,), jnp.float32)

    pltpu.sync_copy(counts_hbm, cntb)
    pltpu.sync_copy(offs_hbm, offb)

    @pl.loop(0, B // 16)
    def _(i):
        mpid[pl.ds(i * 16, 16)] = jnp.zeros((16,), jnp.int32)
        mcell[pl.ds(i * 16, 16)] = jnp.zeros((16,), jnp.int32)

    def read_at(buf, w, rng):
        idx = w * NRANGE + rng
        grp = (idx >> 4) << 4
        vec = buf[pl.ds(grp, 16)]
        sel = jnp.where(iota == (idx & 15), vec, jnp.int32(0))
        return jnp.max(sel)

    def flush(cnt):
        QB = 64
        gathers = [
            pltpu.async_copy(feat_hbm.at[mpid.at[pl.ds(h * QB, QB)]],
                             rows.at[pl.ds(h * QB, QB)], fsem)
            for h in range(B // QB)
        ]

        def row_body(r, carry):
            rbase = mcell[pl.ds(r, 16)][0] * C
            for j in range(C // 16):
                sl = pl.ds(rbase + j * 16, 16)
                acc[sl] = jnp.maximum(acc[sl], rows[r, pl.ds(j * 16, 16)])
            return carry

        for h in range(B // QB):
            gathers[h].wait()
            lax.fori_loop(
                jnp.minimum(cnt, jnp.int32(h * QB)),
                jnp.minimum(cnt, jnp.int32((h + 1) * QB)), row_body, 0)
        return jnp.int32(0)

    for rnd in range(NRANGE // NW):
        rng = wid * (NRANGE // NW) + rnd

        @plsc.parallel_loop(0, GB * C // 16, unroll=8)
        def _(i):
            acc[pl.ds(i * 16, 16)] = zeros16

        heads = [
            pltpu.async_copy(
                buckets_hbm.at[pl.ds(
                    pl.multiple_of(w * WCAP + read_at(offb, w, rng), 8),
                    SEG)],
                segs.at[pl.ds(w * SEG, SEG)], ssem)
            for w in range(NW)
        ]
        for h_ in heads:
            h_.wait()

        def append_chunk(load_fn, rem, cnt):
            nvec = (rem + 15) >> 4

            def vec_body(v, cnt):
                cnt = lax.cond(cnt > B - 16, flush, lambda c: c, cnt)
                vals = load_fn(v)
                mm = (v * 16 + iota) < rem
                plsc.store_compressed(mpid.at[pl.ds(cnt, 16)], vals >> 9,
                                      mask=mm)
                plsc.store_compressed(mcell.at[pl.ds(cnt, 16)], vals & 511,
                                      mask=mm)
                return cnt + jnp.minimum(jnp.int32(16), rem - v * 16)

            return lax.fori_loop(0, nvec, vec_body, cnt)

        def seg_body(w, cnt):
            cw = read_at(cntb, w, rng)
            cnt = append_chunk(lambda v: segs[pl.ds(w * SEG + v * 16, 16)],
                               jnp.minimum(cw, jnp.int32(SEG)), cnt)

            def over_body(k, cnt):
                pltpu.sync_copy(
                    buckets_hbm.at[pl.ds(
                        pl.multiple_of(
                            w * WCAP + read_at(offb, w, rng) + k * SEG, 8),
                        SEG)],
                    xbuf)
                rem = jnp.minimum(cw - k * SEG, jnp.int32(SEG))
                return append_chunk(lambda v: xbuf[pl.ds(v * 16, 16)],
                                    rem, cnt)

            nseg = (cw + SEG - 1) >> 8
            return lax.fori_loop(1, nseg, over_body, cnt)

        cnt = lax.fori_loop(0, NW, seg_body, jnp.int32(0))
        flush(cnt)
        pltpu.sync_copy(acc, out_hbm.at[pl.ds(rng * GB * C, GB * C)])


def kernel(features, points):
    pts_t = jnp.zeros((3, NP), jnp.float32).at[:, :N].set(points.T)
    buckets, counts, offs = _route(pts_t)
    out_flat = _pool(buckets, counts, offs, features)
    return out_flat.reshape(G, C)


# speedup vs baseline: 1.0841x; 1.0015x over previous
import dataclasses
import functools

import jax
import jax.numpy as jnp
from jax import lax
from jax.experimental import pallas as pl
from jax.experimental.pallas import tpu as pltpu
from jax.experimental.pallas import tpu_sc as plsc

W, H, D = 32, 32, 32
G = W * H * D
N, C = 100000, 128

NC, NS = 2, 16
NW = NC * NS
NP = 102400
SL = NP // NW
NV = SL // 16
GB = 512
NRANGE = G // GB
WSORT = SL + NRANGE * 7
SEG = 256
WCAP = WSORT + SEG
B = 256

_mesh = plsc.VectorSubcoreMesh(core_axis_name="c", subcore_axis_name="s")


def _compiler_params():
    cp = pltpu.CompilerParams()
    if "needs_layout_passes" in pltpu.CompilerParams.__dataclass_fields__:
        cp = dataclasses.replace(cp, needs_layout_passes=False)
    return cp


_GDN = lax.GatherDimensionNumbers(
    offset_dims=(), collapsed_slice_dims=(0,), start_index_map=(0,))


def _bcast_lane(vec, lane):
    idx = jnp.broadcast_to(lane.astype(jnp.int32), (16,))[:, None]
    return lax.gather(vec, idx, _GDN, (1,),
                      mode=lax.GatherScatterMode.PROMISE_IN_BOUNDS)


@functools.partial(
    pl.kernel,
    out_type=[
        jax.ShapeDtypeStruct((NW * WCAP,), jnp.int32),
        jax.ShapeDtypeStruct((NW * NRANGE,), jnp.int32),
        jax.ShapeDtypeStruct((NW * NRANGE,), jnp.int32),
    ],
    mesh=_mesh,
    scratch_types=[
        pltpu.VMEM((3, SL), jnp.float32),
        pltpu.VMEM((SL,), jnp.int32),
        pltpu.VMEM((WSORT,), jnp.int32),
        pltpu.VMEM((NRANGE,), jnp.int32),
        pltpu.VMEM((NRANGE,), jnp.int32),
        pltpu.VMEM((NRANGE,), jnp.int32),
    ],
    compiler_params=_compiler_params(),
)
def _route(pts_hbm, buckets_hbm, counts_hbm, offs_hbm,
           pbuf, cellb, stag, hist, offs, cur):
    wid = lax.axis_index("s") * NC + lax.axis_index("c")
    base = wid * SL
    pltpu.sync_copy(pts_hbm.at[:, pl.ds(base, SL)], pbuf)
    iota = lax.iota(jnp.int32, 16)
    ones = jnp.ones((16,), jnp.int32)

    @pl.loop(0, NRANGE // 16)
    def _(i):
        hist[pl.ds(i * 16, 16)] = jnp.zeros((16,), jnp.int32)

    @pl.loop(0, NV)
    def _(v):
        off = v * 16
        x = pbuf[0, pl.ds(off, 16)]
        y = pbuf[1, pl.ds(off, 16)]
        z = pbuf[2, pl.ds(off, 16)]
        ix = jnp.clip((x * W).astype(jnp.int32), 0, W - 1)
        iy = jnp.clip((y * H).astype(jnp.int32), 0, H - 1)
        iz = jnp.clip((z * D).astype(jnp.int32), 0, D - 1)
        flat = (ix * H + iy) * D + iz
        m = (base + off + iota) < N
        cellb[pl.ds(off, 16)] = flat
        plsc.addupdate_scatter(hist, [flat >> 9], ones, mask=m)

    carry = jnp.zeros((16,), jnp.int32)
    for t in range(NRANGE // 16):
        h = hist[pl.ds(t * 16, 16)]
        hp = (h + 7) & jnp.int32(-8)
        inc = plsc.cumsum(hp)
        off_v = inc - hp + carry
        offs[pl.ds(t * 16, 16)] = off_v
        cur[pl.ds(t * 16, 16)] = off_v
        carry = carry + _bcast_lane(inc, jnp.int32(15))

    @pl.loop(0, NV)
    def _(v):
        off = v * 16
        flat = cellb[pl.ds(off, 16)]
        pid = base + off + iota
        m = pid < N
        bucket = flat >> 9
        val = (pid << 9) | (flat & 511)
        dup, _ = plsc.scan_count(bucket, mask=m)
        c0 = plsc.load_gather(cur, [bucket])
        pos = c0 + dup - 1
        plsc.store_scatter(stag, [pos], val, mask=m)
        plsc.addupdate_scatter(cur, [bucket], ones, mask=m)

    pltpu.sync_copy(stag, buckets_hbm.at[pl.ds(wid * WCAP, WSORT)])
    pltpu.sync_copy(hist, counts_hbm.at[pl.ds(wid * NRANGE, NRANGE)])
    pltpu.sync_copy(offs, offs_hbm.at[pl.ds(wid * NRANGE, NRANGE)])


@functools.partial(
    pl.kernel,
    out_type=jax.ShapeDtypeStruct((G * C,), jnp.float32),
    mesh=_mesh,
    scratch_types=[
        pltpu.VMEM((GB * C,), jnp.float32),
        pltpu.VMEM((B, C), jnp.float32),
        pltpu.VMEM((B,), jnp.int32),
        pltpu.VMEM((B + 16,), jnp.int32),
        pltpu.VMEM((NW * SEG,), jnp.int32),
        pltpu.VMEM((SEG,), jnp.int32),
        pltpu.VMEM((NW * NRANGE,), jnp.int32),
        pltpu.VMEM((NW * NRANGE,), jnp.int32),
        pltpu.SemaphoreType.DMA,
        pltpu.SemaphoreType.DMA,
    ],
    compiler_params=_compiler_params(),
)
def _pool(buckets_hbm, counts_hbm, offs_hbm, feat_hbm, out_hbm,
          acc, rows, mpid, mcell, segs, xbuf, cntb, offb, fsem, ssem):
    wid = lax.axis_index("s") * NC + lax.axis_index("c")
    iota = lax.iota(jnp.int32, 16)
    zeros16 = jnp.zeros((16,), jnp.float32)

    pltpu.sync_copy(counts_hbm, cntb)
    pltpu.sync_copy(offs_hbm, offb)

    @pl.loop(0, B // 16)
    def _(i):
        mpid[pl.ds(i * 16, 16)] = jnp.zeros((16,), jnp.int32)
        mcell[pl.ds(i * 16, 16)] = jnp.zeros((16,), jnp.int32)

    def read_at(buf, w, rng):
        idx = w * NRANGE + rng
        grp = (idx >> 4) << 4
        vec = buf[pl.ds(grp, 16)]
        sel = jnp.where(iota == (idx & 15), vec, jnp.int32(0))
        return jnp.max(sel)

    def flush(cnt):
        QB = 32
        gathers = [
            pltpu.async_copy(feat_hbm.at[mpid.at[pl.ds(h * QB, QB)]],
                             rows.at[pl.ds(h * QB, QB)], fsem)
            for h in range(B // QB)
        ]

        def row_body(r, carry):
            rbase = mcell[pl.ds(r, 16)][0] * C
            for j in range(C // 16):
                sl = pl.ds(rbase + j * 16, 16)
                acc[sl] = jnp.maximum(acc[sl], rows[r, pl.ds(j * 16, 16)])
            return carry

        for h in range(B // QB):
            gathers[h].wait()
            lax.fori_loop(
                jnp.minimum(cnt, jnp.int32(h * QB)),
                jnp.minimum(cnt, jnp.int32((h + 1) * QB)), row_body, 0)
        return jnp.int32(0)

    for rnd in range(NRANGE // NW):
        rng = wid * (NRANGE // NW) + rnd

        @plsc.parallel_loop(0, GB * C // 16, unroll=8)
        def _(i):
            acc[pl.ds(i * 16, 16)] = zeros16

        heads = [
            pltpu.async_copy(
                buckets_hbm.at[pl.ds(
                    pl.multiple_of(w * WCAP + read_at(offb, w, rng), 8),
                    SEG)],
                segs.at[pl.ds(w * SEG, SEG)], ssem)
            for w in range(NW)
        ]
        for h_ in heads:
            h_.wait()

        def append_chunk(load_fn, rem, cnt):
            nvec = (rem + 15) >> 4

            def vec_body(v, cnt):
                cnt = lax.cond(cnt > B - 16, flush, lambda c: c, cnt)
                vals = load_fn(v)
                mm = (v * 16 + iota) < rem
                plsc.store_compressed(mpid.at[pl.ds(cnt, 16)], vals >> 9,
                                      mask=mm)
                plsc.store_compressed(mcell.at[pl.ds(cnt, 16)], vals & 511,
                                      mask=mm)
                return cnt + jnp.minimum(jnp.int32(16), rem - v * 16)

            return lax.fori_loop(0, nvec, vec_body, cnt)

        def seg_body(w, cnt):
            cw = read_at(cntb, w, rng)
            cnt = append_chunk(lambda v: segs[pl.ds(w * SEG + v * 16, 16)],
                               jnp.minimum(cw, jnp.int32(SEG)), cnt)

            def over_body(k, cnt):
                pltpu.sync_copy(
                    buckets_hbm.at[pl.ds(
                        pl.multiple_of(
                            w * WCAP + read_at(offb, w, rng) + k * SEG, 8),
                        SEG)],
                    xbuf)
                rem = jnp.minimum(cw - k * SEG, jnp.int32(SEG))
                return append_chunk(lambda v: xbuf[pl.ds(v * 16, 16)],
                                    rem, cnt)

            nseg = (cw + SEG - 1) >> 8
            return lax.fori_loop(1, nseg, over_body, cnt)

        cnt = lax.fori_loop(0, NW, seg_body, jnp.int32(0))
        flush(cnt)
        pltpu.sync_copy(acc, out_hbm.at[pl.ds(rng * GB * C, GB * C)])


def kernel(features, points):
    pts_t = jnp.zeros((3, NP), jnp.float32).at[:, :N].set(points.T)
    buckets, counts, offs = _route(pts_t)
    out_flat = _pool(buckets, counts, offs, features)
    return out_flat.reshape(G, C)
